# baseline (speedup 1.0000x reference)
import functools

import jax
import jax.numpy as jnp
from jax import lax
from jax.experimental import pallas as pl
from jax.experimental.pallas import tpu as pltpu
from jax.experimental.pallas import tpu_sc as plsc

_B = 16384
_F = 16
_HID = 64
_NW = 32
_ROWS_PER_W = _B // _NW
_CHUNK = 128
_NCHUNK = _ROWS_PER_W // _CHUNK
_L = 16

_SCALE = 5.0 - 0.5 + 1.0
_SHIFT = 0.5 - 0.5


def _compute_tile_indices(idx_v, tidx_v):
    for j in range(_NCHUNK):
        def body(g, _, j=j):
            iv = idx_v[j, pl.ds(g * _L, _L)]
            tidx_v[j, pl.ds(g * _L, _L)] = jax.lax.shift_right_logical(iv, 3)
            return 0
        jax.lax.fori_loop(0, _CHUNK // _L, body, 0)


def _extract_chunk(idx_v, j, gat, stage):
    lanes = jax.lax.iota(jnp.int32, _L)

    def body(g, _):
        iv = idx_v[j, pl.ds(g * _L, _L)]
        sub = (iv & 7) * _F
        rows = g * _L + lanes
        off = j * _CHUNK + g * _L
        for c in range(_F):
            stage[c, pl.ds(off, _L)] = plsc.load_gather(gat, [rows, sub + c])
        return 0

    jax.lax.fori_loop(0, _CHUNK // _L, body, 0)


def _gather_body(u_idx, m_idx, u_tab, m_tab, u_out, m_out,
                 u_idx_v, m_idx_v, u_tidx, m_tidx, gat, u_stage, m_stage, sem):
    wid = lax.axis_index("s") * 2 + lax.axis_index("c")
    base = wid * _ROWS_PER_W
    pltpu.sync_copy(u_idx.at[pl.ds(wid * _NCHUNK, _NCHUNK)], u_idx_v)
    pltpu.sync_copy(m_idx.at[pl.ds(wid * _NCHUNK, _NCHUNK)], m_idx_v)
    _compute_tile_indices(u_idx_v, u_tidx)
    _compute_tile_indices(m_idx_v, m_tidx)

    tasks = [(u_tab, u_tidx, u_idx_v, u_stage, j) for j in range(_NCHUNK)]
    tasks += [(m_tab, m_tidx, m_idx_v, m_stage, j) for j in range(_NCHUNK)]

    def fire(t, buf):
        tab, tidx, _, _, j = tasks[t]
        return pltpu.async_copy(tab.at[tidx.at[j]], gat.at[buf], sem)

    handles = {0: fire(0, 0)}
    for t in range(len(tasks)):
        handles[t].wait()
        if t + 1 < len(tasks):
            handles[t + 1] = fire(t + 1, (t + 1) % 2)
        _, _, idx_v, stage, j = tasks[t]
        _extract_chunk(idx_v, j, gat.at[t % 2], stage)

    pltpu.sync_copy(u_stage, u_out.at[:, pl.ds(base, _ROWS_PER_W)])
    pltpu.sync_copy(m_stage, m_out.at[:, pl.ds(base, _ROWS_PER_W)])


@functools.partial(
    pl.kernel,
    out_type=(
        jax.ShapeDtypeStruct((_F, _B), jnp.float32),
        jax.ShapeDtypeStruct((_F, _B), jnp.float32),
    ),
    mesh=plsc.VectorSubcoreMesh(core_axis_name="c", subcore_axis_name="s"),
    compiler_params=pltpu.CompilerParams(needs_layout_passes=False),
    scratch_types=[
        pltpu.VMEM((_NCHUNK, _CHUNK), jnp.int32),
        pltpu.VMEM((_NCHUNK, _CHUNK), jnp.int32),
        pltpu.VMEM((_NCHUNK, _CHUNK), jnp.int32),
        pltpu.VMEM((_NCHUNK, _CHUNK), jnp.int32),
        pltpu.VMEM((2, _CHUNK, 8 * _F), jnp.float32),
        pltpu.VMEM((_F, _ROWS_PER_W), jnp.float32),
        pltpu.VMEM((_F, _ROWS_PER_W), jnp.float32),
        pltpu.SemaphoreType.DMA,
    ],
)
def _gather(u_idx, m_idx, u_tab, m_tab, u_out, m_out,
            u_idx_v, m_idx_v, u_tidx, m_tidx, gat, u_stage, m_stage, sem):
    _gather_body(u_idx, m_idx, u_tab, m_tab, u_out, m_out,
                 u_idx_v, m_idx_v, u_tidx, m_tidx, gat, u_stage, m_stage, sem)


_BLK = 2048


def _mlp_body(u_ref, m_ref, w1a_ref, w1b_ref, b1_ref, w2_ref, b2_ref, o_ref):
    h = jnp.dot(w1a_ref[...], u_ref[...],
                preferred_element_type=jnp.float32,
                precision=lax.Precision.HIGHEST)
    h = h + jnp.dot(w1b_ref[...], m_ref[...],
                    preferred_element_type=jnp.float32,
                    precision=lax.Precision.HIGHEST)
    h = jnp.maximum(h + b1_ref[...], 0.0)
    t = jnp.sum(h * w2_ref[...], axis=0, keepdims=True) + b2_ref[...]
    o_ref[...] = jax.nn.sigmoid(t) * _SCALE + _SHIFT


def _mlp(u_embt, m_embt, w1at, w1bt, b1, w2, b2):
    grid = (_B // _BLK,)
    return pl.pallas_call(
        _mlp_body,
        grid=grid,
        in_specs=[
            pl.BlockSpec((_F, _BLK), lambda i: (0, i)),
            pl.BlockSpec((_F, _BLK), lambda i: (0, i)),
            pl.BlockSpec((_HID, _F), lambda i: (0, 0)),
            pl.BlockSpec((_HID, _F), lambda i: (0, 0)),
            pl.BlockSpec((_HID, 1), lambda i: (0, 0)),
            pl.BlockSpec((_HID, 1), lambda i: (0, 0)),
            pl.BlockSpec((1, 1), lambda i: (0, 0)),
        ],
        out_specs=pl.BlockSpec((1, _BLK), lambda i: (0, i)),
        out_shape=jax.ShapeDtypeStruct((1, _B), jnp.float32),
    )(u_embt, m_embt, w1at, w1bt, b1, w2, b2)


def kernel(user, movie, u_table, m_table, W1, b1, W2, b2):
    u_idx = user.astype(jnp.int32).reshape(_NW * _NCHUNK, _CHUNK)
    m_idx = movie.astype(jnp.int32).reshape(_NW * _NCHUNK, _CHUNK)
    u_tiles = u_table.reshape(-1, 8 * _F)
    m_tiles = m_table.reshape(-1, 8 * _F)
    u_embt, m_embt = _gather(u_idx, m_idx, u_tiles, m_tiles)
    out = _mlp(u_embt, m_embt, W1[:_F].T, W1[_F:].T,
               b1.reshape(_HID, 1), W2.reshape(_HID, 1), b2.reshape(1, 1))
    return out.reshape(_B, 1)
